# baseline (speedup 1.0000x reference)
import jax
import jax.numpy as jnp
from jax import lax
from jax.experimental import pallas as pl
from jax.experimental.pallas import tpu as pltpu
from jax.experimental.pallas import tpu_sc as plsc

N = 10000
E = 320000
D = 128
H = 4
C = 32
W = 144

NEG_SLOPE = 0.2


def _head_expand_matrix(dtype=jnp.float32):
    h_ids = lax.broadcasted_iota(jnp.int32, (H, D), 0)
    d_ids = lax.broadcasted_iota(jnp.int32, (H, D), 1)
    return (d_ids // C == h_ids).astype(dtype)


def _proj_body(x_ref, wl_ref, bl_ref, wr_ref, br_ref, attf_ref,
               xl_ref, xr_ref, selfex_ref):
    x = x_ref[...]
    xl = jnp.dot(x, wl_ref[...], preferred_element_type=jnp.float32) + bl_ref[...]
    xr = jnp.dot(x, wr_ref[...], preferred_element_type=jnp.float32) + br_ref[...]
    xl_ref[...] = xl
    xr_ref[...] = xr
    z = xl + xr
    t = jnp.maximum(z, 0.0) + NEG_SLOPE * jnp.minimum(z, 0.0)
    p = t * attf_ref[...]
    bexp = _head_expand_matrix()
    alpha = jnp.dot(p, bexp.T, preferred_element_type=jnp.float32)
    selfex_ref[...] = jnp.exp(alpha)


def _run_proj(x, W_l, b_l, W_r, b_r, att_flat):
    blk = 2000
    grid = (N // blk,)
    return pl.pallas_call(
        _proj_body,
        grid=grid,
        in_specs=[
            pl.BlockSpec((blk, D), lambda i: (i, 0)),
            pl.BlockSpec((D, D), lambda i: (0, 0)),
            pl.BlockSpec((D,), lambda i: (0,)),
            pl.BlockSpec((D, D), lambda i: (0, 0)),
            pl.BlockSpec((D,), lambda i: (0,)),
            pl.BlockSpec((D,), lambda i: (0,)),
        ],
        out_specs=[
            pl.BlockSpec((blk, D), lambda i: (i, 0)),
            pl.BlockSpec((blk, D), lambda i: (i, 0)),
            pl.BlockSpec((blk, H), lambda i: (i, 0)),
        ],
        out_shape=[
            jax.ShapeDtypeStruct((N, D), jnp.float32),
            jax.ShapeDtypeStruct((N, D), jnp.float32),
            jax.ShapeDtypeStruct((N, H), jnp.float32),
        ],
    )(x, W_l, b_l, W_r, b_r, att_flat)


def _combine_body(acc_ref, den_ref, x_ref, xl_ref, selfex_ref,
                  biasg_ref, gnw_ref, gnb_ref, gnms_ref, out_ref):
    num = acc_ref[0, :N] + acc_ref[1, :N]
    den4 = den_ref[0, :N] + den_ref[1, :N] + selfex_ref[...]
    bexp = _head_expand_matrix()
    den = jnp.dot(den4, bexp, preferred_element_type=jnp.float32)
    self_exp = jnp.dot(selfex_ref[...], bexp, preferred_element_type=jnp.float32)
    num = num + xl_ref[...] * self_exp
    gat = num / (den + 1e-16) + biasg_ref[...]
    h1 = gat + x_ref[...]
    h1 = jnp.where(h1 > 0.0, h1, jnp.exp(jnp.minimum(h1, 0.0)) - 1.0)
    mean = jnp.sum(h1, axis=0, keepdims=True) * (1.0 / N)
    ctr = h1 - mean * gnms_ref[...]
    var = jnp.sum(ctr * ctr, axis=0, keepdims=True) * (1.0 / N)
    std = jnp.sqrt(var + 1e-5)
    out_ref[...] = gnw_ref[...] * ctr / std + gnb_ref[...]


def _run_combine(acc, den, x, xl, selfex, bias_gat, gn_w, gn_b, gn_ms):
    return pl.pallas_call(
        _combine_body,
        out_shape=jax.ShapeDtypeStruct((N, D), jnp.float32),
    )(acc, den, x, xl, selfex, bias_gat, gn_w, gn_b, gn_ms)


NC = 2
NS = 16
L = 16
EK = 40
E_PER_SUB = E // (NC * NS)
NBLK = E_PER_SUB // EK
N_PAD = 10240
NDEN = 320
N_ACC = N_PAD + 384
ROWS_PER_SUB = N_ACC // NS
ZROWS = 8


def _lane_perm(v, idx):
    return lax.gather(
        v, idx[:, None],
        lax.GatherDimensionNumbers(
            offset_dims=(), collapsed_slice_dims=(0,), start_index_map=(0,)),
        (1,), mode=lax.GatherScatterMode.PROMISE_IN_BOUNDS)


def _edge_body(xl_hbm, xr_hbm, src_hbm, dst_hbm, att_hbm, out_hbm,
               srcv0, srcv1, dstv0, dstv1, dsts0, dsts1, divr0, divr1,
               u0, u1, v0, v1, msg0, msg1, md0, md1,
               zbuf, attv, acc, gsem0, gsem1, ssem0, ssem1):
    c = lax.axis_index("c")
    s = lax.axis_index("s")

    srcv = (srcv0, srcv1)
    dstv = (dstv0, dstv1)
    dsts = (dsts0, dsts1)
    divr = (divr0, divr1)
    u_b = (u0, u1)
    v_b = (v0, v1)
    msg_b = (msg0, msg1)
    md_b = (md0, md1)
    gsem = (gsem0, gsem1)
    ssem = (ssem0, ssem1)

    pltpu.sync_copy(att_hbm, attv)

    def _zero_zbuf(i, _):
        for j in range(D // L):
            zbuf[i, pl.ds(L * j, L)] = jnp.zeros((L,), jnp.float32)
        return _
    lax.fori_loop(0, ZROWS, _zero_zbuf, None)

    def _zero_acc(i, _):
        pltpu.sync_copy(zbuf, acc.at[pl.ds(s * ROWS_PER_SUB + i * ZROWS, ZROWS)])
        return _
    lax.fori_loop(0, ROWS_PER_SUB // ZROWS, _zero_acc, None)
    plsc.subcore_barrier()

    av = [attv[pl.ds(L * j, L)] for j in range(D // L)]
    lanes = lax.iota(jnp.int32, L)
    x8, x4, x2, x1 = [lanes ^ k for k in (8, 4, 2, 1)]
    m0 = (lanes >> 2) == 0
    m1 = (lanes >> 2) == 1
    m2 = (lanes >> 2) == 2
    bidx = [jnp.full((L,), 4 * h, jnp.int32) for h in range(H)]
    lmap = 4 * (lanes & 3)
    grp = [(L * j + lanes) >> 2 for j in range(D // L)]
    zero = jnp.zeros((L,), jnp.float32)

    ebase = (c * NS + s) * E_PER_SUB

    def _load_idx(i, b):
        base = jnp.minimum(ebase + i * EK, E - EK)
        pltpu.sync_copy(src_hbm.at[pl.ds(base, EK)], srcv[b])
        pltpu.sync_copy(dst_hbm.at[pl.ds(base, EK)], dstv[b])

    def _issue_gather(b):
        pltpu.async_copy(xl_hbm.at[srcv[b]], u_b[b], gsem[b])
        pltpu.async_copy(xr_hbm.at[dstv[b]], v_b[b], gsem[b])

    def _wait_gather(b):
        pltpu.make_async_copy(xl_hbm.at[srcv[b]], u_b[b], gsem[b]).wait()
        pltpu.make_async_copy(xr_hbm.at[dstv[b]], v_b[b], gsem[b]).wait()

    def _issue_scatter(b):
        pltpu.async_copy(msg_b[b], acc.at[dsts[b]], ssem[b], add=True)
        pltpu.async_copy(md_b[b], acc.at[divr[b]], ssem[b], add=True)

    def _wait_scatter(b):
        pltpu.make_async_copy(msg_b[b], acc.at[dsts[b]], ssem[b]).wait()
        pltpu.make_async_copy(md_b[b], acc.at[divr[b]], ssem[b]).wait()

    def _compute(b):
        ub, vb, mb, db = u_b[b], v_b[b], msg_b[b], md_b[b]
        dv, ds_, dr = dstv[b], dsts[b], divr[b]

        for off in (0, 16, 24):
            d16 = dv[pl.ds(off, L)]
            ds_[pl.ds(off, L)] = d16
            dr[pl.ds(off, L)] = (d16 >> 5) + N_PAD

        def _hgroup(hg, _):
            off = jnp.minimum(4 * hg, EK - L)
            loff = 4 * hg - off
            dst16 = dv[pl.ds(off, L)]
            for k in range(4):
                e = 4 * hg + k
                t = []
                for h in range(H):
                    sv = None
                    for j in (2 * h, 2 * h + 1):
                        z = ub[e, pl.ds(L * j, L)] + vb[e, pl.ds(L * j, L)]
                        p = jnp.maximum(z, NEG_SLOPE * z) * av[j]
                        sv = p if sv is None else sv + p
                    sv = sv + _lane_perm(sv, x8)
                    sv = sv + _lane_perm(sv, x4)
                    t.append(sv)
                w = jnp.where(m0, t[0],
                              jnp.where(m1, t[1],
                                        jnp.where(m2, t[2], t[3])))
                w = w + _lane_perm(w, x2)
                w = w + _lane_perm(w, x1)
                exw = jnp.exp(w)
                exb = [_lane_perm(exw, bidx[h]) for h in range(H)]
                for j in range(D // L):
                    mb[e, pl.ds(L * j, L)] = ub[e, pl.ds(L * j, L)] * exb[j // 2]
                exlane = _lane_perm(exw, lmap)
                dstb = _lane_perm(dst16, jnp.full((L,), 1, jnp.int32) * (loff + k))
                slot = dstb & 31
                for j in range(D // L):
                    db[e, pl.ds(L * j, L)] = jnp.where(
                        grp[j] == slot, exlane, zero)
            return _
        lax.fori_loop(0, EK // 4, _hgroup, None)

    _load_idx(0, 0)
    _issue_gather(0)

    _load_idx(1, 1)
    _issue_gather(1)
    _wait_gather(0)
    _compute(0)
    _issue_scatter(0)
    _load_idx(2, 0)
    _issue_gather(0)
    _wait_gather(1)
    _compute(1)
    _issue_scatter(1)

    def _pair(p, _):
        for b in (0, 1):
            i = 2 * p + b
            _load_idx(i + 1, 1 - b)
            _issue_gather(1 - b)
            _wait_gather(b)
            _wait_scatter(b)
            _compute(b)
            _issue_scatter(b)
        return _
    lax.fori_loop(1, NBLK // 2, _pair, None)

    _wait_gather(0)
    _wait_scatter(0)
    _wait_scatter(1)

    plsc.subcore_barrier()
    pltpu.sync_copy(acc.at[pl.ds(s * ROWS_PER_SUB, ROWS_PER_SUB)],
                    out_hbm.at[c, pl.ds(s * ROWS_PER_SUB, ROWS_PER_SUB)])


def _run_edges(xl, xr, src, dst, att_flat):
    import functools
    mesh = plsc.VectorSubcoreMesh(core_axis_name="c", subcore_axis_name="s")
    f = functools.partial(
        pl.kernel,
        out_type=jax.ShapeDtypeStruct((NC, N_ACC, D), jnp.float32),
        mesh=mesh,
        scratch_types=(
            [pltpu.VMEM((EK,), jnp.int32)] * 8
            + [pltpu.VMEM((EK, D), jnp.float32)] * 8
            + [
                pltpu.VMEM((ZROWS, D), jnp.float32),
                pltpu.VMEM((D,), jnp.float32),
                pltpu.VMEM_SHARED((N_ACC, D), jnp.float32),
                pltpu.SemaphoreType.DMA,
                pltpu.SemaphoreType.DMA,
                pltpu.SemaphoreType.DMA,
                pltpu.SemaphoreType.DMA,
            ]
        ),
    )(_edge_body)
    return f(xl, xr, src, dst, att_flat)


def kernel(x, W_l, b_l, W_r, b_r, att, bias_gat, gn_weight, gn_bias,
           gn_mean_scale, edge_index, batch):
    att_flat = att.reshape(D)
    src = edge_index[0]
    dst = edge_index[1]
    xl, xr, selfex = _run_proj(x, W_l, b_l, W_r, b_r, att_flat)
    accall = _run_edges(xl, xr, src, dst, att_flat)
    accn = accall[:, :N_PAD]
    accd = accall[:, N_PAD:N_PAD + NDEN].reshape(NC, NDEN * D // H, H)
    out = _run_combine(accn, accd, x, xl, selfex, bias_gat,
                       gn_weight, gn_bias, gn_mean_scale)
    return out


# speedup vs baseline: 2.6053x; 2.6053x over previous
import jax
import jax.numpy as jnp
from jax import lax
from jax.experimental import pallas as pl
from jax.experimental.pallas import tpu as pltpu
from jax.experimental.pallas import tpu_sc as plsc

N = 10000
E = 320000
D = 128
H = 4
C = 32
W = 144

NEG_SLOPE = 0.2


def _head_expand_matrix(dtype=jnp.float32):
    h_ids = lax.broadcasted_iota(jnp.int32, (H, D), 0)
    d_ids = lax.broadcasted_iota(jnp.int32, (H, D), 1)
    return (d_ids // C == h_ids).astype(dtype)


def _proj_body(x_ref, wl_ref, bl_ref, wr_ref, br_ref, attf_ref,
               xl_ref, xr_ref, selfex_ref):
    x = x_ref[...]
    xl = jnp.dot(x, wl_ref[...], preferred_element_type=jnp.float32) + bl_ref[...]
    xr = jnp.dot(x, wr_ref[...], preferred_element_type=jnp.float32) + br_ref[...]
    xl_ref[...] = xl
    xr_ref[...] = xr
    z = xl + xr
    t = jnp.maximum(z, 0.0) + NEG_SLOPE * jnp.minimum(z, 0.0)
    p = t * attf_ref[...]
    bexp = _head_expand_matrix()
    alpha = jnp.dot(p, bexp.T, preferred_element_type=jnp.float32)
    selfex_ref[...] = jnp.exp(alpha)


def _run_proj(x, W_l, b_l, W_r, b_r, att_flat):
    blk = 2000
    grid = (N // blk,)
    return pl.pallas_call(
        _proj_body,
        grid=grid,
        in_specs=[
            pl.BlockSpec((blk, D), lambda i: (i, 0)),
            pl.BlockSpec((D, D), lambda i: (0, 0)),
            pl.BlockSpec((D,), lambda i: (0,)),
            pl.BlockSpec((D, D), lambda i: (0, 0)),
            pl.BlockSpec((D,), lambda i: (0,)),
            pl.BlockSpec((D,), lambda i: (0,)),
        ],
        out_specs=[
            pl.BlockSpec((blk, D), lambda i: (i, 0)),
            pl.BlockSpec((blk, D), lambda i: (i, 0)),
            pl.BlockSpec((blk, H), lambda i: (i, 0)),
        ],
        out_shape=[
            jax.ShapeDtypeStruct((N, D), jnp.float32),
            jax.ShapeDtypeStruct((N, D), jnp.float32),
            jax.ShapeDtypeStruct((N, H), jnp.float32),
        ],
    )(x, W_l, b_l, W_r, b_r, att_flat)


def _combine_body(acc_ref, den_ref, x_ref, xl_ref, selfex_ref,
                  biasg_ref, gnw_ref, gnb_ref, gnms_ref, out_ref):
    num = acc_ref[0, :N] + acc_ref[1, :N]
    den4 = den_ref[0, :N] + den_ref[1, :N] + selfex_ref[...]
    bexp = _head_expand_matrix()
    den = jnp.dot(den4, bexp, preferred_element_type=jnp.float32)
    self_exp = jnp.dot(selfex_ref[...], bexp, preferred_element_type=jnp.float32)
    num = num + xl_ref[...] * self_exp
    gat = num / (den + 1e-16) + biasg_ref[...]
    h1 = gat + x_ref[...]
    h1 = jnp.where(h1 > 0.0, h1, jnp.exp(jnp.minimum(h1, 0.0)) - 1.0)
    mean = jnp.sum(h1, axis=0, keepdims=True) * (1.0 / N)
    ctr = h1 - mean * gnms_ref[...]
    var = jnp.sum(ctr * ctr, axis=0, keepdims=True) * (1.0 / N)
    std = jnp.sqrt(var + 1e-5)
    out_ref[...] = gnw_ref[...] * ctr / std + gnb_ref[...]


def _run_combine(acc, den, x, xl, selfex, bias_gat, gn_w, gn_b, gn_ms):
    return pl.pallas_call(
        _combine_body,
        out_shape=jax.ShapeDtypeStruct((N, D), jnp.float32),
    )(acc, den, x, xl, selfex, bias_gat, gn_w, gn_b, gn_ms)


NC = 2
NS = 16
L = 16
EK = 40
E_PER_SUB = E // (NC * NS)
NBLK = E_PER_SUB // EK
N_PAD = 10240
NDEN = 320
N_ACC = N_PAD + 384
ROWS_PER_SUB = N_ACC // NS
ZROWS = 8


def _lane_perm(v, idx):
    return lax.gather(
        v, idx[:, None],
        lax.GatherDimensionNumbers(
            offset_dims=(), collapsed_slice_dims=(0,), start_index_map=(0,)),
        (1,), mode=lax.GatherScatterMode.PROMISE_IN_BOUNDS)


def _edge_body(xl_hbm, xr_hbm, src_hbm, dst_hbm, att_hbm, out_hbm,
               srcv0, srcv1, dstv0, dstv1, dsts0, dsts1, divr0, divr1,
               dvc0, dvc1,
               u0, u1, v0, v1, msg0, msg1, md0, md1,
               zbuf, attv, acc, gsem0, gsem1, ssem0, ssem1, isem0, isem1):
    c = lax.axis_index("c")
    s = lax.axis_index("s")

    srcv = (srcv0, srcv1)
    dstv = (dstv0, dstv1)
    dsts = (dsts0, dsts1)
    divr = (divr0, divr1)
    dvc = (dvc0, dvc1)
    u_b = (u0, u1)
    v_b = (v0, v1)
    msg_b = (msg0, msg1)
    md_b = (md0, md1)
    gsem = (gsem0, gsem1)
    ssem = (ssem0, ssem1)
    isem = (isem0, isem1)

    pltpu.sync_copy(att_hbm, attv)

    def _zero_zbuf(i, _):
        for j in range(D // L):
            zbuf[i, pl.ds(L * j, L)] = jnp.zeros((L,), jnp.float32)
        return _
    lax.fori_loop(0, ZROWS, _zero_zbuf, None)

    def _zero_acc(i, _):
        pltpu.sync_copy(zbuf, acc.at[pl.ds(s * ROWS_PER_SUB + i * ZROWS, ZROWS)])
        return _
    lax.fori_loop(0, ROWS_PER_SUB // ZROWS, _zero_acc, None)
    plsc.subcore_barrier()

    av = [attv[pl.ds(L * j, L)] for j in range(D // L)]
    lanes = lax.iota(jnp.int32, L)
    x8, x4, x2, x1 = [lanes ^ k for k in (8, 4, 2, 1)]
    m0 = (lanes >> 2) == 0
    m1 = (lanes >> 2) == 1
    m2 = (lanes >> 2) == 2
    bidx = [jnp.full((L,), 4 * h, jnp.int32) for h in range(H)]
    lmap = 4 * (lanes & 3)
    grp = [(L * j + lanes) >> 2 for j in range(D // L)]
    zero = jnp.zeros((L,), jnp.float32)

    ebase = (c * NS + s) * E_PER_SUB

    def _load_idx(i, b):
        base = jnp.minimum(ebase + i * EK, E - EK)
        pltpu.sync_copy(src_hbm.at[pl.ds(base, EK)], srcv[b])
        pltpu.sync_copy(dst_hbm.at[pl.ds(base, EK)], dstv[b])

    def _issue_idx(i, b):
        base = jnp.minimum(ebase + i * EK, E - EK)
        pltpu.async_copy(src_hbm.at[pl.ds(base, EK)], srcv[b], isem[b])
        pltpu.async_copy(dst_hbm.at[pl.ds(base, EK)], dstv[b], isem[b])

    def _wait_idx(i, b):
        base = jnp.minimum(ebase + i * EK, E - EK)
        pltpu.make_async_copy(src_hbm.at[pl.ds(base, EK)], srcv[b], isem[b]).wait()
        pltpu.make_async_copy(dst_hbm.at[pl.ds(base, EK)], dstv[b], isem[b]).wait()

    def _issue_gather(b):
        pltpu.async_copy(xl_hbm.at[srcv[b]], u_b[b], gsem[b])
        pltpu.async_copy(xr_hbm.at[dstv[b]], v_b[b], gsem[b])

    def _wait_gather(b):
        pltpu.make_async_copy(xl_hbm.at[srcv[b]], u_b[b], gsem[b]).wait()
        pltpu.make_async_copy(xr_hbm.at[dstv[b]], v_b[b], gsem[b]).wait()

    def _issue_scatter(b):
        pltpu.async_copy(msg_b[b], acc.at[dsts[b]], ssem[b], add=True)
        pltpu.async_copy(md_b[b], acc.at[divr[b]], ssem[b], add=True)

    def _wait_scatter(b):
        pltpu.make_async_copy(msg_b[b], acc.at[dsts[b]], ssem[b]).wait()
        pltpu.make_async_copy(md_b[b], acc.at[divr[b]], ssem[b]).wait()

    def _snap(b):
        dv, ds_, dr, dc = dstv[b], dsts[b], divr[b], dvc[b]
        for off in (0, 16, 24):
            d16 = dv[pl.ds(off, L)]
            ds_[pl.ds(off, L)] = d16
            dr[pl.ds(off, L)] = (d16 >> 5) + N_PAD
            dc[pl.ds(off, L)] = d16

    def _compute(b):
        ub, vb, mb, db = u_b[b], v_b[b], msg_b[b], md_b[b]
        dv = dvc[b]

        def _hgroup(hg, _):
            off = jnp.minimum(8 * hg, EK - L)
            loff = 8 * hg - off
            dst16 = dv[pl.ds(off, L)]
            for k in range(8):
                e = 8 * hg + k
                t = []
                for h in range(H):
                    sv = None
                    for j in (2 * h, 2 * h + 1):
                        z = ub[e, pl.ds(L * j, L)] + vb[e, pl.ds(L * j, L)]
                        p = jnp.maximum(z, NEG_SLOPE * z) * av[j]
                        sv = p if sv is None else sv + p
                    sv = sv + _lane_perm(sv, x8)
                    sv = sv + _lane_perm(sv, x4)
                    t.append(sv)
                w = jnp.where(m0, t[0],
                              jnp.where(m1, t[1],
                                        jnp.where(m2, t[2], t[3])))
                w = w + _lane_perm(w, x2)
                w = w + _lane_perm(w, x1)
                exw = jnp.exp(w)
                exb = [_lane_perm(exw, bidx[h]) for h in range(H)]
                for j in range(D // L):
                    mb[e, pl.ds(L * j, L)] = ub[e, pl.ds(L * j, L)] * exb[j // 2]
                exlane = _lane_perm(exw, lmap)
                dstb = _lane_perm(dst16, jnp.full((L,), 1, jnp.int32) * (loff + k))
                slot = dstb & 31
                for j in range(D // L):
                    db[e, pl.ds(L * j, L)] = jnp.where(
                        grp[j] == slot, exlane, zero)
            return _
        lax.fori_loop(0, EK // 8, _hgroup, None)

    _load_idx(0, 0)
    _issue_gather(0)

    _load_idx(1, 1)
    _issue_gather(1)
    _wait_gather(0)
    _snap(0)
    _issue_idx(2, 0)
    _compute(0)
    _issue_scatter(0)
    _wait_idx(2, 0)
    _issue_gather(0)
    _wait_gather(1)
    _snap(1)
    _issue_idx(3, 1)
    _compute(1)
    _issue_scatter(1)

    def _pair(p, _):
        for b in (0, 1):
            i = 2 * p + b
            _wait_idx(i + 1, 1 - b)
            _issue_gather(1 - b)
            _wait_gather(b)
            _wait_scatter(b)
            _snap(b)
            _issue_idx(i + 2, b)
            _compute(b)
            _issue_scatter(b)
        return _
    lax.fori_loop(1, NBLK // 2, _pair, None)

    _wait_gather(0)
    _wait_idx(NBLK + 1, 1)
    _wait_scatter(0)
    _wait_scatter(1)

    plsc.subcore_barrier()
    pltpu.sync_copy(acc.at[pl.ds(s * ROWS_PER_SUB, ROWS_PER_SUB)],
                    out_hbm.at[c, pl.ds(s * ROWS_PER_SUB, ROWS_PER_SUB)])


def _run_edges(xl, xr, src, dst, att_flat):
    import functools
    mesh = plsc.VectorSubcoreMesh(core_axis_name="c", subcore_axis_name="s")
    f = functools.partial(
        pl.kernel,
        out_type=jax.ShapeDtypeStruct((NC, N_ACC, D), jnp.float32),
        mesh=mesh,
        scratch_types=(
            [pltpu.VMEM((EK,), jnp.int32)] * 10
            + [pltpu.VMEM((EK, D), jnp.float32)] * 8
            + [
                pltpu.VMEM((ZROWS, D), jnp.float32),
                pltpu.VMEM((D,), jnp.float32),
                pltpu.VMEM_SHARED((N_ACC, D), jnp.float32),
                pltpu.SemaphoreType.DMA,
                pltpu.SemaphoreType.DMA,
                pltpu.SemaphoreType.DMA,
                pltpu.SemaphoreType.DMA,
                pltpu.SemaphoreType.DMA,
                pltpu.SemaphoreType.DMA,
            ]
        ),
    )(_edge_body)
    return f(xl, xr, src, dst, att_flat)


def kernel(x, W_l, b_l, W_r, b_r, att, bias_gat, gn_weight, gn_bias,
           gn_mean_scale, edge_index, batch):
    att_flat = att.reshape(D)
    src = edge_index[0]
    dst = edge_index[1]
    xl, xr, selfex = _run_proj(x, W_l, b_l, W_r, b_r, att_flat)
    accall = _run_edges(xl, xr, src, dst, att_flat)
    accn = accall[:, :N_PAD]
    accd = accall[:, N_PAD:N_PAD + NDEN].reshape(NC, NDEN * D // H, H)
    out = _run_combine(accn, accd, x, xl, selfex, bias_gat,
                       gn_weight, gn_bias, gn_mean_scale)
    return out
